# Initial kernel scaffold; baseline (speedup 1.0000x reference)
import jax
import jax.numpy as jnp
from jax.experimental import pallas as pl


def kernel(inputs, table):
    raise NotImplementedError("write your pallas kernel here")


# speedup vs baseline: 1.4583x; 1.4583x over previous
import functools

import jax
import jax.numpy as jnp
from jax import lax
from jax.experimental import pallas as pl
from jax.experimental.pallas import tpu as pltpu
from jax.experimental.pallas import tpu_sc as plsc

B = 4096
L = 200
EMB = 32
BFLAT = B * L

NC = 2
NS = 16
NW = NC * NS

PER_W = BFLAT // NW
CHUNK = 1024
NCHUNK = PER_W // CHUNK
SUB = 128
NSUB = CHUNK // SUB


def _sc_embedding_lookup(idx_flat, table):
    mesh = plsc.VectorSubcoreMesh(
        core_axis_name="c", subcore_axis_name="s",
        num_cores=NC, num_subcores=NS)

    @functools.partial(
        pl.kernel,
        mesh=mesh,
        out_type=jax.ShapeDtypeStruct((BFLAT, EMB), jnp.float32),
        scratch_types=[
            pltpu.VMEM((CHUNK,), jnp.int32),
            pltpu.VMEM((CHUNK, EMB), jnp.float32),
            pltpu.SemaphoreType.DMA,
        ],
        compiler_params=pltpu.CompilerParams(use_tc_tiling_on_sc=False),
    )
    def k(idx_hbm, table_hbm, out_hbm, idx_v, rows_v, sem):
        wid = lax.axis_index("s") * NC + lax.axis_index("c")
        base = wid * PER_W

        def body(c, carry):
            off = base + c * CHUNK
            pltpu.sync_copy(idx_hbm.at[pl.ds(off, CHUNK)], idx_v)
            copies = [
                pltpu.async_copy(
                    table_hbm.at[idx_v.at[pl.ds(j * SUB, SUB)]],
                    rows_v.at[pl.ds(j * SUB, SUB)],
                    sem)
                for j in range(NSUB)
            ]
            for cp in copies:
                cp.wait()
            pltpu.sync_copy(rows_v, out_hbm.at[pl.ds(off, CHUNK)])
            return carry

        lax.fori_loop(0, NCHUNK, body, 0)

    return k(idx_flat, table)


def kernel(inputs, table):
    idx_flat = inputs.reshape(-1).astype(jnp.int32)
    out = _sc_embedding_lookup(idx_flat, table)
    return out.reshape(B, L, EMB)


# speedup vs baseline: 1.4985x; 1.0276x over previous
import functools

import jax
import jax.numpy as jnp
from jax import lax
from jax.experimental import pallas as pl
from jax.experimental.pallas import tpu as pltpu
from jax.experimental.pallas import tpu_sc as plsc

B = 4096
L = 200
EMB = 32
BFLAT = B * L

NC = 2
NS = 16
NW = NC * NS

PER_W = BFLAT // NW
CHUNK = 1280
NCHUNK = PER_W // CHUNK
NPAIR = NCHUNK // 2
SUB = 128
NSUB = CHUNK // SUB


def _sc_embedding_lookup(idx_flat, table):
    mesh = plsc.VectorSubcoreMesh(
        core_axis_name="c", subcore_axis_name="s",
        num_cores=NC, num_subcores=NS)

    @functools.partial(
        pl.kernel,
        mesh=mesh,
        out_type=jax.ShapeDtypeStruct((BFLAT, EMB), jnp.float32),
        scratch_types=[
            pltpu.VMEM((2, CHUNK), jnp.int32),
            pltpu.VMEM((2, CHUNK, EMB), jnp.float32),
            pltpu.SemaphoreType.DMA,
            pltpu.SemaphoreType.DMA,
        ],
        compiler_params=pltpu.CompilerParams(use_tc_tiling_on_sc=False),
    )
    def k(idx_hbm, table_hbm, out_hbm, idx_v, rows_v, gsem, wsem):
        wid = lax.axis_index("s") * NC + lax.axis_index("c")
        base = wid * PER_W

        def fire_chunk(c, p):
            off = base + c * CHUNK
            pltpu.sync_copy(idx_hbm.at[pl.ds(off, CHUNK)], idx_v.at[p])
            return [
                pltpu.async_copy(
                    table_hbm.at[idx_v.at[p].at[pl.ds(j * SUB, SUB)]],
                    rows_v.at[p].at[pl.ds(j * SUB, SUB)],
                    gsem)
                for j in range(NSUB)
            ]

        def wait_writeback(p):
            pltpu.make_async_copy(
                rows_v.at[p], out_hbm.at[pl.ds(base, CHUNK)], wsem).wait()

        def writeback(c, p):
            off = base + c * CHUNK
            pltpu.async_copy(rows_v.at[p], out_hbm.at[pl.ds(off, CHUNK)],
                             wsem)

        def body(t, carry):
            a = 2 * t

            @pl.when(t > 0)
            def _():
                wait_writeback(0)

            ga = fire_chunk(a, 0)

            @pl.when(t > 0)
            def _():
                wait_writeback(1)

            gb = fire_chunk(a + 1, 1)
            for cp in ga:
                cp.wait()
            writeback(a, 0)
            for cp in gb:
                cp.wait()
            writeback(a + 1, 1)
            return carry

        lax.fori_loop(0, NPAIR, body, 0)
        wait_writeback(0)
        wait_writeback(1)

    return k(idx_flat, table)


def kernel(inputs, table):
    idx_flat = inputs.reshape(-1).astype(jnp.int32)
    out = _sc_embedding_lookup(idx_flat, table)
    return out.reshape(B, L, EMB)


# speedup vs baseline: 1.5754x; 1.0513x over previous
import functools

import jax
import jax.numpy as jnp
from jax import lax
from jax.experimental import pallas as pl
from jax.experimental.pallas import tpu as pltpu
from jax.experimental.pallas import tpu_sc as plsc

B = 4096
L = 200
EMB = 32

NC = 2
NS = 16
NW = NC * NS

BB = B // NW
K = 4
NBATCH = L // K
NPAIR = NBATCH // 2


def _sc_embedding_lookup(idx_lb, table):
    mesh = plsc.VectorSubcoreMesh(
        core_axis_name="c", subcore_axis_name="s",
        num_cores=NC, num_subcores=NS)

    @functools.partial(
        pl.kernel,
        mesh=mesh,
        out_type=jax.ShapeDtypeStruct((L, B, EMB), jnp.float32),
        scratch_types=[
            pltpu.VMEM((L, BB), jnp.int32),
            pltpu.VMEM((2, K, BB, EMB), jnp.float32),
            pltpu.SemaphoreType.DMA,
            pltpu.SemaphoreType.DMA,
        ],
        compiler_params=pltpu.CompilerParams(use_tc_tiling_on_sc=False),
    )
    def k(idx_hbm, table_hbm, out_hbm, idx_v, rows_v, gsem, wsem):
        wid = lax.axis_index("s") * NC + lax.axis_index("c")
        b0 = wid * BB
        pltpu.sync_copy(idx_hbm.at[:, pl.ds(b0, BB)], idx_v)

        def fire(t, p):
            return [
                pltpu.async_copy(
                    table_hbm.at[idx_v.at[K * t + j]],
                    rows_v.at[p].at[j],
                    gsem)
                for j in range(K)
            ]

        def writeback(t, p):
            pltpu.async_copy(
                rows_v.at[p],
                out_hbm.at[pl.ds(K * t, K), pl.ds(b0, BB)],
                wsem)

        def wait_writeback(p):
            pltpu.make_async_copy(
                rows_v.at[p],
                out_hbm.at[pl.ds(0, K), pl.ds(b0, BB)],
                wsem).wait()

        def body(t, carry):
            a = 2 * t

            @pl.when(t > 0)
            def _():
                wait_writeback(0)

            ga = fire(a, 0)

            @pl.when(t > 0)
            def _():
                wait_writeback(1)

            gb = fire(a + 1, 1)
            for cp in ga:
                cp.wait()
            writeback(a, 0)
            for cp in gb:
                cp.wait()
            writeback(a + 1, 1)
            return carry

        lax.fori_loop(0, NPAIR, body, 0)
        wait_writeback(0)
        wait_writeback(1)

    return k(idx_lb, table)


def kernel(inputs, table):
    idx_lb = jnp.swapaxes(inputs, 0, 1).astype(jnp.int32)
    out = _sc_embedding_lookup(idx_lb, table)
    return jnp.transpose(out, (1, 0, 2))
